# baseline (speedup 1.0000x reference)
import jax
import jax.numpy as jnp
from jax import lax
from jax.experimental import pallas as pl
from jax.experimental.pallas import tpu as pltpu
from jax.experimental.pallas import tpu_sc as plsc

_NUM_USERS = 1000
_VOCAB = 2000
_D = 128
_B = 4096
_NW = 32
_BPW = _B // _NW


def _sc_gather_body(user_hbm, item_hbm, tab_hbm,
                    gu_out, gi_out,
                    uidx, iidx, gu_v, gi_v, sem_i, sem_g, sem_w):
    wid = lax.axis_index("s") * 2 + lax.axis_index("c")
    base = wid * _BPW
    ci0 = pltpu.async_copy(user_hbm.at[pl.ds(base, _BPW)], uidx, sem_i)
    ci1 = pltpu.async_copy(item_hbm.at[pl.ds(base, _BPW)], iidx, sem_i)
    ci0.wait()
    ci1.wait()
    for j in range(_BPW // 16):
        iidx[pl.ds(j * 16, 16)] = iidx[pl.ds(j * 16, 16)] + _NUM_USERS
    c0 = pltpu.async_copy(tab_hbm.at[uidx], gu_v, sem_g)
    c1 = pltpu.async_copy(tab_hbm.at[iidx], gi_v, sem_g)
    c0.wait()
    w0 = pltpu.async_copy(gu_v, gu_out.at[pl.ds(base, _BPW)], sem_w)
    c1.wait()
    w1 = pltpu.async_copy(gi_v, gi_out.at[pl.ds(base, _BPW)], sem_w)
    w0.wait()
    w1.wait()


def _sc_gather(user, item, tab):
    mesh = plsc.VectorSubcoreMesh(core_axis_name="c", subcore_axis_name="s")
    f = pl.kernel(
        _sc_gather_body, mesh=mesh,
        compiler_params=pltpu.CompilerParams(needs_layout_passes=False),
        out_type=tuple(
            jax.ShapeDtypeStruct((_B, _D), jnp.float32) for _ in range(2)),
        scratch_types=[
            pltpu.VMEM((_BPW,), jnp.int32),
            pltpu.VMEM((_BPW,), jnp.int32),
            pltpu.VMEM((_BPW, _D), jnp.float32),
            pltpu.VMEM((_BPW, _D), jnp.float32),
            pltpu.SemaphoreType.DMA,
            pltpu.SemaphoreType.DMA,
            pltpu.SemaphoreType.DMA,
        ],
    )
    return f(user, item, tab)


def _unpack(ui):
    lo = lax.bitcast_convert_type(ui << 16, jnp.float32)
    hi = lax.bitcast_convert_type(ui & jnp.int32(-65536), jnp.float32)
    return lo, hi


def _mlp_body(gu, gi, g, t, W0ev, W0od, W0gt, b0, W1, b1, Wfh, Wfwev, Wfwod,
              bf, b_wide, Wfw, out):
    i32 = jnp.int32
    gub = lax.bitcast_convert_type(gu[...], i32)
    gib = lax.bitcast_convert_type(gi[...], i32)
    emb_a, emb_b = _unpack(gub[:, 0:64] | gib[:, 0:64])
    wua, wub = _unpack(gub[:, 64:128])
    wia, wib = _unpack(gib[:, 64:128])
    x = (jnp.dot(emb_a, W0ev[...], preferred_element_type=jnp.float32)
         + jnp.dot(emb_b, W0od[...], preferred_element_type=jnp.float32)
         + jnp.dot(jnp.concatenate([g[...], t[...]], axis=1), W0gt[...],
                   preferred_element_type=jnp.float32)
         + b0[...])
    h0 = jnp.maximum(x, 0.0)
    h1 = jnp.maximum(
        jnp.dot(h0, W1[...], preferred_element_type=jnp.float32) + b1[...], 0.0)
    wide_bias = jnp.sum(b_wide[...] * Wfw[...][:, 0]) + bf[0]
    logits = (jnp.dot(h1, Wfh[...], preferred_element_type=jnp.float32)
              + jnp.dot(wua + wia, Wfwev[...], preferred_element_type=jnp.float32)
              + jnp.dot(wub + wib, Wfwod[...], preferred_element_type=jnp.float32)
              + wide_bias)
    out[...] = logits


def _mlp(gu, gi, genre, tag, W0ev, W0od, W0gt, b0, W1, b1, Wfh, Wfwev, Wfwod,
         bf, b_wide, Wfw):
    nb = 4
    blk = _B // nb
    rep = lambda shape: pl.BlockSpec(shape, lambda i: (0,) * len(shape))
    row = lambda d: pl.BlockSpec((blk, d), lambda i: (i, 0))
    return pl.pallas_call(
        _mlp_body,
        grid=(nb,),
        in_specs=[
            row(_D), row(_D), row(20), row(100),
            rep((64, 256)), rep((64, 256)), rep((120, 256)), rep((256,)),
            rep((256, 128)), rep((128,)),
            rep((128, 1)), rep((64, 1)), rep((64, 1)),
            rep((1,)), rep((128,)), rep((128, 1)),
        ],
        out_specs=row(1),
        out_shape=jax.ShapeDtypeStruct((_B, 1), jnp.float32),
    )(gu, gi, genre, tag, W0ev, W0od, W0gt, b0, W1, b1, Wfh, Wfwev, Wfwod,
      bf, b_wide, Wfw)


def kernel(user, item, genre, tag, W_wide, b_wide, user_table, item_table,
           W0, b0, W1, b1, Wf, bf):
    user = user.astype(jnp.int32)
    item = item.astype(jnp.int32)
    bf16 = jnp.bfloat16
    zeros = jnp.zeros_like(user_table)
    emb2 = jnp.concatenate([
        jnp.concatenate([user_table, zeros], axis=1),
        jnp.concatenate([zeros, item_table], axis=1),
    ], axis=0)
    both = jnp.concatenate([emb2.astype(bf16), W_wide.astype(bf16)], axis=1)
    tab = lax.bitcast_convert_type(
        both.reshape(_VOCAB, _D, 2), jnp.float32)
    gu, gi = _sc_gather(user, item, tab)
    return _mlp(gu, gi, genre, tag,
                W0[0:128:2, :], W0[1:128:2, :], W0[128:248, :], b0,
                W1, b1,
                Wf[0:128, :], Wf[128:256:2, :], Wf[129:256:2, :],
                bf, b_wide, Wf[128:256, :])


# speedup vs baseline: 1.1338x; 1.1338x over previous
import jax
import jax.numpy as jnp
from jax import lax
from jax.experimental import pallas as pl
from jax.experimental.pallas import tpu as pltpu
from jax.experimental.pallas import tpu_sc as plsc

_NUM_USERS = 1000
_D = 128
_B = 4096
_NW = 32
_BPW = _B // _NW


def _sc_gather_body(user_hbm, item_hbm, emb2_hbm, ww_hbm,
                    gu_out, gi_out, wu_out, wi_out,
                    uidx, iidx, gu_v, gi_v, wu_v, wi_v, sem):
    wid = lax.axis_index("s") * 2 + lax.axis_index("c")
    base = wid * _BPW
    pltpu.sync_copy(user_hbm.at[pl.ds(base, _BPW)], uidx)
    pltpu.sync_copy(item_hbm.at[pl.ds(base, _BPW)], iidx)
    for j in range(_BPW // 16):
        iidx[pl.ds(j * 16, 16)] = iidx[pl.ds(j * 16, 16)] + _NUM_USERS
    c0 = pltpu.async_copy(emb2_hbm.at[uidx], gu_v, sem)
    c1 = pltpu.async_copy(emb2_hbm.at[iidx], gi_v, sem)
    c2 = pltpu.async_copy(ww_hbm.at[uidx], wu_v, sem)
    c3 = pltpu.async_copy(ww_hbm.at[iidx], wi_v, sem)
    c0.wait()
    c1.wait()
    c2.wait()
    c3.wait()
    pltpu.sync_copy(gu_v, gu_out.at[pl.ds(base, _BPW)])
    pltpu.sync_copy(gi_v, gi_out.at[pl.ds(base, _BPW)])
    pltpu.sync_copy(wu_v, wu_out.at[pl.ds(base, _BPW)])
    pltpu.sync_copy(wi_v, wi_out.at[pl.ds(base, _BPW)])


def _sc_gather(user, item, emb2, W_wide):
    mesh = plsc.VectorSubcoreMesh(core_axis_name="c", subcore_axis_name="s")
    f = pl.kernel(
        _sc_gather_body, mesh=mesh,
        out_type=tuple(
            jax.ShapeDtypeStruct((_B, _D), jnp.float32) for _ in range(4)),
        scratch_types=[
            pltpu.VMEM((_BPW,), jnp.int32),
            pltpu.VMEM((_BPW,), jnp.int32),
            pltpu.VMEM((_BPW, _D), jnp.float32),
            pltpu.VMEM((_BPW, _D), jnp.float32),
            pltpu.VMEM((_BPW, _D), jnp.float32),
            pltpu.VMEM((_BPW, _D), jnp.float32),
            pltpu.SemaphoreType.DMA,
        ],
    )
    return f(user, item, emb2, W_wide)


def _mlp_body(gu, gi, g, t, wu, wi, W0, b0, W1, b1, Wf, bf, b_wide, out):
    emb = gu[...] + gi[...]
    x = (jnp.dot(emb, W0[0:128, :], preferred_element_type=jnp.float32)
         + jnp.dot(g[...], W0[128:148, :], preferred_element_type=jnp.float32)
         + jnp.dot(t[...], W0[148:248, :], preferred_element_type=jnp.float32)
         + b0[...])
    h0 = jnp.maximum(x, 0.0)
    h1 = jnp.maximum(
        jnp.dot(h0, W1[...], preferred_element_type=jnp.float32) + b1[...], 0.0)
    wide = wu[...] + wi[...] + b_wide[...]
    logits = (jnp.dot(h1, Wf[0:128, :], preferred_element_type=jnp.float32)
              + jnp.dot(wide, Wf[128:256, :], preferred_element_type=jnp.float32)
              + bf[...])
    out[...] = logits


def _mlp(gu, gi, genre, tag, wu, wi, W0, b0, W1, b1, Wf, bf, b_wide):
    nb = 4
    blk = _B // nb
    rep = lambda shape: pl.BlockSpec(shape, lambda i: (0,) * len(shape))
    row = lambda d: pl.BlockSpec((blk, d), lambda i: (i, 0))
    return pl.pallas_call(
        _mlp_body,
        grid=(nb,),
        in_specs=[
            row(_D), row(_D), row(20), row(100), row(_D), row(_D),
            rep((248, 256)), rep((256,)), rep((256, 128)), rep((128,)),
            rep((256, 1)), rep((1,)), rep((128,)),
        ],
        out_specs=row(1),
        out_shape=jax.ShapeDtypeStruct((_B, 1), jnp.float32),
    )(gu, gi, genre, tag, wu, wi, W0, b0, W1, b1, Wf, bf, b_wide)


def kernel(user, item, genre, tag, W_wide, b_wide, user_table, item_table,
           W0, b0, W1, b1, Wf, bf):
    user = user.astype(jnp.int32)
    item = item.astype(jnp.int32)
    zeros = jnp.zeros_like(user_table)
    emb2 = jnp.concatenate([
        jnp.concatenate([user_table, zeros], axis=1),
        jnp.concatenate([zeros, item_table], axis=1),
    ], axis=0)
    gu, gi, wu, wi = _sc_gather(user, item, emb2, W_wide)
    return _mlp(gu, gi, genre, tag, wu, wi, W0, b0, W1, b1, Wf, bf, b_wide)
